# baseline (speedup 1.0000x reference)
import functools

import jax
import jax.numpy as jnp
from jax import lax
from jax.experimental import pallas as pl
from jax.experimental.pallas import tpu as pltpu
from jax.experimental.pallas import tpu_sc as plsc

B = 1024
D = 128
V = 100000

_NC, _NS = 2, 16
_NW = _NC * _NS
_BPW = B // _NW

_sc_mesh = plsc.VectorSubcoreMesh(
    core_axis_name="c", subcore_axis_name="s",
    num_cores=_NC, num_subcores=_NS)


@functools.partial(
    pl.kernel,
    out_type=jax.ShapeDtypeStruct((B, D), jnp.float32),
    mesh=_sc_mesh,
    scratch_types=[
        pltpu.VMEM((_BPW,), jnp.int32),
        pltpu.VMEM((_BPW, D), jnp.float32),
        pltpu.SemaphoreType.DMA,
    ],
)
def _sc_gather(idx_hbm, table_hbm, out_hbm, idx_v, rows_v, sem):
    wid = lax.axis_index("s") * _NC + lax.axis_index("c")
    base = wid * _BPW
    pltpu.sync_copy(idx_hbm.at[pl.ds(base, _BPW)], idx_v)
    pltpu.async_copy(table_hbm.at[idx_v], rows_v, sem).wait()
    pltpu.sync_copy(rows_v, out_hbm.at[pl.ds(base, _BPW)])


_VT = 2048
_NVT = (V + _VT - 1) // _VT


def _mm_body(emb_ref, w_ref, b_ref, out_ref):
    out_ref[...] = (
        jnp.dot(emb_ref[...], w_ref[...], preferred_element_type=jnp.float32)
        + b_ref[...]
    )


_mm = pl.pallas_call(
    _mm_body,
    grid=(_NVT,),
    in_specs=[
        pl.BlockSpec((B, D), lambda v: (0, 0)),
        pl.BlockSpec((D, _VT), lambda v: (0, v)),
        pl.BlockSpec((1, _VT), lambda v: (0, v)),
    ],
    out_specs=pl.BlockSpec((B, _VT), lambda v: (0, v)),
    out_shape=jax.ShapeDtypeStruct((B, V), jnp.float32),
)


@jax.jit
def kernel(target_idx, emb_table, W, b):
    embed = _sc_gather(target_idx, emb_table)
    return _mm(embed, W, b.reshape(1, V))
